# baseline (speedup 1.0000x reference)
import functools

import jax
import jax.numpy as jnp
from jax import lax
from jax.experimental import pallas as pl
from jax.experimental.pallas import tpu as pltpu
from jax.experimental.pallas import tpu_sc as plsc

HIDDEN = 384
NORM_EPS = 1e-05
LANES = 16
NV = HIDDEN // LANES

NC = 2
NS = 16
NW = NC * NS

CHUNK = 64
NBUF = 4
P1CHUNK = 48


def _tree_sum(xs):
    xs = list(xs)
    while len(xs) > 1:
        nxt = [a + b for a, b in zip(xs[0::2], xs[1::2])]
        if len(xs) % 2:
            nxt.append(xs[-1])
        xs = nxt
    return xs[0]


def _rsqrt_vec(x):
    i = plsc.bitcast(x, jnp.int32)
    i = jnp.int32(0x5F3759DF) - lax.shift_right_logical(i, 1)
    y = plsc.bitcast(i, jnp.float32)
    for _ in range(3):
        y = y * (1.5 - 0.5 * x * y * y)
    return y


def _lane_sum(v, perms):
    for p in perms:
        v = v + jnp.take_along_axis(v, p, axis=0)
    return v


def _stats_rows(rows_ref, mean_t, inv_t, r0, nrows):
    iota = lax.iota(jnp.int32, LANES)
    perms = [iota ^ s for s in (8, 4, 2, 1)]

    @plsc.parallel_loop(0, nrows // LANES, step=1)
    def _(gidx):
        macc = jnp.zeros((LANES,), jnp.float32)
        iacc = jnp.zeros((LANES,), jnp.float32)
        for u in range(LANES):
            r = gidx * LANES + u
            vs = [rows_ref[r, pl.ds(LANES * j, LANES)] for j in range(NV)]
            s = _lane_sum(_tree_sum(vs), perms)
            q = _lane_sum(_tree_sum([v * v for v in vs]), perms)
            mean = s * (1.0 / HIDDEN)
            var = jnp.maximum(q * (1.0 / HIDDEN) - mean * mean, 0.0) + NORM_EPS
            inv = _rsqrt_vec(var)
            sel = iota == u
            macc = jnp.where(sel, mean, macc)
            iacc = jnp.where(sel, inv, iacc)
        mean_t[pl.ds(r0 + gidx * LANES, LANES)] = macc
        inv_t[pl.ds(r0 + gidx * LANES, LANES)] = iacc


def _normalize_chunk(buf, idx_v, i, mean_t, inv_t, ws):
    @plsc.parallel_loop(0, CHUNK // LANES, step=1)
    def _(g):
        ids16 = idx_v[i, pl.ds(g * LANES, LANES)]
        m16 = plsc.load_gather(mean_t, [ids16])
        i16 = plsc.load_gather(inv_t, [ids16])
        for u in range(LANES):
            pu = jnp.full((LANES,), u, jnp.int32)
            mb = jnp.take_along_axis(m16, pu, axis=0)
            ib = jnp.take_along_axis(i16, pu, axis=0)
            tok = g * LANES + u
            for j in range(NV):
                v = buf[tok, pl.ds(LANES * j, LANES)]
                buf[tok, pl.ds(LANES * j, LANES)] = (v - mb) * (ib * ws[j])


def _body(nchunk, vocab, ids_hbm, table_hbm, w_hbm, out_hbm,
          idx_v, w_v, mean_t, inv_t, sp_mean, sp_inv, buf0, buf1, buf2, buf3,
          gsem0, gsem1, gsem2, gsem3, ssem0, ssem1, ssem2, ssem3):
    sid = lax.axis_index("s")
    wid = sid * NC + lax.axis_index("c")
    base = wid * (nchunk * CHUNK)
    bufs = (buf0, buf1, buf2, buf3)
    gsems = (gsem0, gsem1, gsem2, gsem3)
    ssems = (ssem0, ssem1, ssem2, ssem3)

    pltpu.sync_copy(w_hbm, w_v)
    pltpu.sync_copy(ids_hbm.at[wid], idx_v)
    ws = [w_v[pl.ds(LANES * j, LANES)] for j in range(NV)]

    for b in range(NBUF - 1):
        pltpu.async_copy(table_hbm.at[idx_v.at[b]], bufs[b], gsems[b])

    rows_per_tile = vocab // NS
    r_base = sid * rows_per_tile
    for c in range(rows_per_tile // P1CHUNK):
        r0 = r_base + c * P1CHUNK
        pltpu.sync_copy(table_hbm.at[pl.ds(r0, P1CHUNK)],
                        buf3.at[pl.ds(0, P1CHUNK)])
        _stats_rows(buf3, mean_t, inv_t, r0, P1CHUNK)
    pltpu.sync_copy(mean_t.at[pl.ds(r_base, rows_per_tile)],
                    sp_mean.at[pl.ds(r_base, rows_per_tile)])
    pltpu.sync_copy(inv_t.at[pl.ds(r_base, rows_per_tile)],
                    sp_inv.at[pl.ds(r_base, rows_per_tile)])
    plsc.subcore_barrier()
    pltpu.sync_copy(sp_mean, mean_t)
    pltpu.sync_copy(sp_inv, inv_t)

    def group_body(g, carry):
        for b in range(NBUF):
            i = g * NBUF + b
            pltpu.make_async_copy(
                table_hbm.at[idx_v.at[i]], bufs[b], gsems[b]).wait()
            _normalize_chunk(bufs[b], idx_v, i, mean_t, inv_t, ws)
            pltpu.async_copy(
                bufs[b], out_hbm.at[pl.ds(base + i * CHUNK, CHUNK)], ssems[b])

            bj = (b + NBUF - 1) % NBUF

            def drain_scatter():
                pltpu.make_async_copy(
                    bufs[bj], out_hbm.at[pl.ds(base, CHUNK)], ssems[bj]).wait()

            def issue_gather():
                pltpu.async_copy(
                    table_hbm.at[idx_v.at[i + NBUF - 1]], bufs[bj], gsems[bj])

            if b == 0:
                pl.when(g > 0)(drain_scatter)
                issue_gather()
            else:
                drain_scatter()
                pl.when(g < nchunk // NBUF - 1)(issue_gather)
        return carry

    lax.fori_loop(0, nchunk // NBUF, group_body, 0)

    blast = (nchunk - 1) % NBUF
    pltpu.make_async_copy(
        bufs[blast], out_hbm.at[pl.ds(base, CHUNK)], ssems[blast]).wait()


@jax.jit
def kernel(input_ids, tok_embeddings, norm_weight):
    batch, seq = input_ids.shape
    total = batch * seq
    vocab = tok_embeddings.shape[0]
    assert total % (NW * CHUNK) == 0
    nchunk = total // (NW * CHUNK)
    assert nchunk % NBUF == 0
    assert vocab % (NS * P1CHUNK) == 0 and P1CHUNK % LANES == 0

    ids = input_ids.reshape(NW, nchunk, CHUNK).astype(jnp.int32)

    mesh = plsc.VectorSubcoreMesh(
        core_axis_name="c", subcore_axis_name="s", num_cores=NC,
        num_subcores=NS)
    out = pl.kernel(
        functools.partial(_body, nchunk, vocab),
        out_type=jax.ShapeDtypeStruct((total, HIDDEN), jnp.float32),
        mesh=mesh,
        compiler_params=pltpu.CompilerParams(needs_layout_passes=False),
        scratch_types=[
            pltpu.VMEM((nchunk, CHUNK), jnp.int32),
            pltpu.VMEM((HIDDEN,), jnp.float32),
            pltpu.VMEM((vocab,), jnp.float32),
            pltpu.VMEM((vocab,), jnp.float32),
            pltpu.VMEM_SHARED((vocab,), jnp.float32),
            pltpu.VMEM_SHARED((vocab,), jnp.float32),
            pltpu.VMEM((CHUNK, HIDDEN), jnp.float32),
            pltpu.VMEM((CHUNK, HIDDEN), jnp.float32),
            pltpu.VMEM((CHUNK, HIDDEN), jnp.float32),
            pltpu.VMEM((CHUNK, HIDDEN), jnp.float32),
            pltpu.SemaphoreType.DMA,
            pltpu.SemaphoreType.DMA,
            pltpu.SemaphoreType.DMA,
            pltpu.SemaphoreType.DMA,
            pltpu.SemaphoreType.DMA,
            pltpu.SemaphoreType.DMA,
            pltpu.SemaphoreType.DMA,
            pltpu.SemaphoreType.DMA,
        ],
    )(ids, tok_embeddings, norm_weight)
    return out.reshape(batch, seq, HIDDEN)
